# baseline (speedup 1.0000x reference)
import functools

import jax
import jax.numpy as jnp
from jax import lax
from jax.experimental import pallas as pl
from jax.experimental.pallas import tpu as pltpu
from jax.experimental.pallas import tpu_sc as plsc

B = 16384
NCOL = 26
VOCAB = 100000
EDIM = 16
DIN = 200 + 1 + NCOL * EDIM

NC = 2
NS = 16
NW = NC * NS
B_TOT = B * NCOL
B_PER_W = B_TOT // NW
CHUNK = 1664
NCHUNK = B_PER_W // CHUNK


def _sc_gather(table_flat, idx_flat):
    mesh = plsc.VectorSubcoreMesh(core_axis_name="c", subcore_axis_name="s")

    @functools.partial(
        pl.kernel,
        mesh=mesh,
        out_type=jax.ShapeDtypeStruct((B_TOT, EDIM), jnp.float32),
        scratch_types=[
            pltpu.VMEM((CHUNK,), jnp.int32),
            pltpu.VMEM((CHUNK, EDIM), jnp.float32),
            pltpu.SemaphoreType.DMA,
        ],
        compiler_params=pltpu.CompilerParams(use_tc_tiling_on_sc=False),
    )
    def gather_k(table_hbm, idx_hbm, out_hbm, idx_v, rows_v, sem):
        wid = lax.axis_index("s") * NC + lax.axis_index("c")
        base = wid * B_PER_W
        for i in range(NCHUNK):
            off = base + i * CHUNK
            pltpu.sync_copy(idx_hbm.at[pl.ds(off, CHUNK)], idx_v)
            pltpu.async_copy(table_hbm.at[idx_v], rows_v, sem).wait()
            pltpu.sync_copy(rows_v, out_hbm.at[pl.ds(off, CHUNK)])

    return gather_k(table_flat, idx_flat)


BS = 512


def _mlp_body(meta_ref, nla_ref, emb_ref, w1m_ref, w1n_ref, w1e_ref,
              b1_ref, w2_ref, b2_ref, out_ref, pred_ref):
    m = meta_ref[...]
    n = nla_ref[...]
    e = emb_ref[...]
    out_ref[...] = jnp.concatenate([m, n, e], axis=1)
    h = jnp.dot(m, w1m_ref[...], preferred_element_type=jnp.float32)
    h = h + jnp.dot(e, w1e_ref[...], preferred_element_type=jnp.float32)
    h = h + n * w1n_ref[...]
    h = jnp.maximum(h + b1_ref[...], 0.0)
    z = jnp.dot(h, w2_ref[...], preferred_element_type=jnp.float32) + b2_ref[...]
    pred_ref[...] = 1.0 / (1.0 + jnp.exp(-z))


def _tc_mlp(meta, nla, emb, w1m, w1n, w1e, b1, w2, b2):
    grid = (B // BS,)
    blk = lambda r, c: pl.BlockSpec((r, c), lambda i: (i, 0))
    full = lambda r, c: pl.BlockSpec((r, c), lambda i: (0, 0))
    return pl.pallas_call(
        _mlp_body,
        grid=grid,
        in_specs=[
            blk(BS, 200), blk(BS, 1), blk(BS, NCOL * EDIM),
            full(200, 20), full(1, 20), full(NCOL * EDIM, 20),
            full(1, 20), full(20, 1), full(1, 1),
        ],
        out_specs=[blk(BS, DIN), blk(BS, 1)],
        out_shape=[
            jax.ShapeDtypeStruct((B, DIN), jnp.float32),
            jax.ShapeDtypeStruct((B, 1), jnp.float32),
        ],
    )(meta, nla, emb, w1m, w1n, w1e, b1, w2, b2)


def kernel(meta_features, nla, components, tables, W1, b1, W2, b2):
    table_flat = tables.reshape(NCOL * VOCAB, EDIM)
    col_off = (jnp.arange(NCOL, dtype=jnp.int32) * VOCAB)[None, :]
    idx_flat = (components.astype(jnp.int32) + col_off).reshape(B_TOT)

    emb_flat = _sc_gather(table_flat, idx_flat)
    return emb_flat
    emb = emb_flat.reshape(B, NCOL * EDIM)

    w1m = W1[0:200]
    w1n = W1[200:201]
    w1e = W1[201:DIN]
    embedding, pred = _tc_mlp(meta_features, nla, emb, w1m, w1n, w1e,
                              b1.reshape(1, 20), W2, b2.reshape(1, 1))
    return (embedding, pred)


# speedup vs baseline: 3.0089x; 3.0089x over previous
import functools

import jax
import jax.numpy as jnp
from jax import lax
from jax.experimental import pallas as pl
from jax.experimental.pallas import tpu as pltpu
from jax.experimental.pallas import tpu_sc as plsc

B = 16384
NCOL = 26
VOCAB = 100000
EDIM = 16
DIN = 200 + 1 + NCOL * EDIM

NC = 2
NS = 16
NW = NC * NS
B_TOT = B * NCOL
B_PER_W = B_TOT // NW
CHUNK = 1664
NCHUNK = B_PER_W // CHUNK

VBLK = 12544
NBLK = 8
VPAD = NBLK * VBLK
PR = VBLK // 8


def _xpose_body(t_ref, out_ref):
    x = t_ref[0]
    lv = lax.broadcasted_iota(jnp.int32, (EDIM, VBLK), 1) + pl.program_id(1) * VBLK
    x = jnp.where(lv < VOCAB, x, 0.0)
    xs = jnp.concatenate([x[:, s * PR:(s + 1) * PR] for s in range(8)], axis=0)
    out_ref[...] = jnp.transpose(xs)[None]


def _tc_transpose(tT):
    return pl.pallas_call(
        _xpose_body,
        grid=(NCOL, NBLK),
        in_specs=[pl.BlockSpec((1, EDIM, VBLK), lambda c, j: (c, 0, j))],
        out_specs=pl.BlockSpec((1, PR, 128), lambda c, j: (c, j, 0)),
        out_shape=jax.ShapeDtypeStruct((NCOL, NBLK * PR, 128), jnp.float32),
    )(tT)


def _sc_gather(table_flat, idx_flat):
    mesh = plsc.VectorSubcoreMesh(core_axis_name="c", subcore_axis_name="s")

    @functools.partial(
        pl.kernel,
        mesh=mesh,
        out_type=jax.ShapeDtypeStruct((B_TOT, EDIM), jnp.float32),
        scratch_types=[
            pltpu.VMEM((CHUNK,), jnp.int32),
            pltpu.VMEM((CHUNK, EDIM), jnp.float32),
            pltpu.SemaphoreType.DMA,
        ],
        compiler_params=pltpu.CompilerParams(use_tc_tiling_on_sc=False),
    )
    def gather_k(table_hbm, idx_hbm, out_hbm, idx_v, rows_v, sem):
        wid = lax.axis_index("s") * NC + lax.axis_index("c")
        base = wid * B_PER_W
        for i in range(NCHUNK):
            off = base + i * CHUNK
            pltpu.sync_copy(idx_hbm.at[pl.ds(off, CHUNK)], idx_v)
            pltpu.async_copy(table_hbm.at[idx_v], rows_v, sem).wait()
            pltpu.sync_copy(rows_v, out_hbm.at[pl.ds(off, CHUNK)])

    return gather_k(table_flat, idx_flat)


BS = 512


def _mlp_body(meta_ref, nla_ref, emb_ref, w1m_ref, w1n_ref, w1e_ref,
              b1_ref, w2_ref, b2_ref, out_ref, pred_ref):
    m = meta_ref[...]
    n = nla_ref[...]
    e = emb_ref[...]
    out_ref[...] = jnp.concatenate([m, n, e], axis=1)
    h = jnp.dot(m, w1m_ref[...], preferred_element_type=jnp.float32)
    h = h + jnp.dot(e, w1e_ref[...], preferred_element_type=jnp.float32)
    h = h + n * w1n_ref[...]
    h = jnp.maximum(h + b1_ref[...], 0.0)
    z = jnp.dot(h, w2_ref[...], preferred_element_type=jnp.float32) + b2_ref[...]
    pred_ref[...] = 1.0 / (1.0 + jnp.exp(-z))


def _tc_mlp(meta, nla, emb, w1m, w1n, w1e, b1, w2, b2):
    grid = (B // BS,)
    blk = lambda r, c: pl.BlockSpec((r, c), lambda i: (i, 0))
    full = lambda r, c: pl.BlockSpec((r, c), lambda i: (0, 0))
    return pl.pallas_call(
        _mlp_body,
        grid=grid,
        in_specs=[
            blk(BS, 200), blk(BS, 1), blk(BS, NCOL * EDIM),
            full(200, 20), full(1, 20), full(NCOL * EDIM, 20),
            full(1, 20), full(20, 1), full(1, 1),
        ],
        out_specs=[blk(BS, DIN), blk(BS, 1)],
        out_shape=[
            jax.ShapeDtypeStruct((B, DIN), jnp.float32),
            jax.ShapeDtypeStruct((B, 1), jnp.float32),
        ],
    )(meta, nla, emb, w1m, w1n, w1e, b1, w2, b2)


def kernel(meta_features, nla, components, tables, W1, b1, W2, b2):
    tT = tables.transpose(0, 2, 1)
    t128 = _tc_transpose(tT)
    table_flat = t128.reshape(NCOL * VPAD, EDIM)

    comp = components.astype(jnp.int32)
    col_off = (jnp.arange(NCOL, dtype=jnp.int32) * VPAD)[None, :]
    j, vl = comp // VBLK, comp % VBLK
    packed = (j * PR + vl % PR) * 8 + vl // PR
    idx_flat = (packed + col_off).reshape(B_TOT)

    emb_flat = _sc_gather(table_flat, idx_flat)
    emb = emb_flat.reshape(B, NCOL * EDIM)

    w1m = W1[0:200]
    w1n = W1[200:201]
    w1e = W1[201:DIN]
    embedding, pred = _tc_mlp(meta_features, nla, emb, w1m, w1n, w1e,
                              b1.reshape(1, 20), W2, b2.reshape(1, 1))
    return (embedding, pred)


# speedup vs baseline: 3.9651x; 1.3178x over previous
import functools

import jax
import jax.numpy as jnp
from jax import lax
from jax.experimental import pallas as pl
from jax.experimental.pallas import tpu as pltpu
from jax.experimental.pallas import tpu_sc as plsc

B = 16384
NCOL = 26
VOCAB = 100000
EDIM = 16
DIN = 200 + 1 + NCOL * EDIM

NC = 2
NS = 16
NW = NC * NS
B_TOT = B * NCOL
B_PER_W = B_TOT // NW
CHUNK = 1664
NCHUNK = B_PER_W // CHUNK

VBLK = 50176
NBLK = 2
VPAD = NBLK * VBLK
PR = VBLK // 8


def _xpose_body(t_ref, out_ref):
    x = t_ref[0]
    lv = lax.broadcasted_iota(jnp.int32, (EDIM, VBLK), 1) + pl.program_id(1) * VBLK
    x = jnp.where(lv < VOCAB, x, 0.0)
    xs = jnp.concatenate([x[:, s * PR:(s + 1) * PR] for s in range(8)], axis=0)
    out_ref[...] = jnp.transpose(xs)[None]


def _tc_transpose(tT):
    return pl.pallas_call(
        _xpose_body,
        grid=(NCOL, NBLK),
        in_specs=[pl.BlockSpec((1, EDIM, VBLK), lambda c, j: (c, 0, j))],
        out_specs=pl.BlockSpec((1, PR, 128), lambda c, j: (c, j, 0)),
        out_shape=jax.ShapeDtypeStruct((NCOL, NBLK * PR, 128), jnp.float32),
    )(tT)


def _sc_gather(table_flat, idx_flat):
    mesh = plsc.VectorSubcoreMesh(core_axis_name="c", subcore_axis_name="s")

    @functools.partial(
        pl.kernel,
        mesh=mesh,
        out_type=jax.ShapeDtypeStruct((B_TOT, EDIM), jnp.float32),
        scratch_types=[
            pltpu.VMEM((CHUNK,), jnp.int32),
            pltpu.VMEM((CHUNK, EDIM), jnp.float32),
            pltpu.SemaphoreType.DMA,
        ],
        compiler_params=pltpu.CompilerParams(use_tc_tiling_on_sc=False),
    )
    def gather_k(table_hbm, idx_hbm, out_hbm, idx_v, rows_v, sem):
        wid = lax.axis_index("s") * NC + lax.axis_index("c")
        base = wid * B_PER_W
        for i in range(NCHUNK):
            off = base + i * CHUNK
            pltpu.sync_copy(idx_hbm.at[pl.ds(off, CHUNK)], idx_v)
            pltpu.async_copy(table_hbm.at[idx_v], rows_v, sem).wait()
            pltpu.sync_copy(rows_v, out_hbm.at[pl.ds(off, CHUNK)])

    return gather_k(table_flat, idx_flat)


BS = 512


def _mlp_body(meta_ref, nla_ref, emb_ref, w1m_ref, w1n_ref, w1e_ref,
              b1_ref, w2_ref, b2_ref, out_ref, pred_ref):
    m = meta_ref[...]
    n = nla_ref[...]
    e = emb_ref[...]
    out_ref[...] = jnp.concatenate([m, n, e], axis=1)
    h = jnp.dot(m, w1m_ref[...], preferred_element_type=jnp.float32)
    h = h + jnp.dot(e, w1e_ref[...], preferred_element_type=jnp.float32)
    h = h + n * w1n_ref[...]
    h = jnp.maximum(h + b1_ref[...], 0.0)
    z = jnp.dot(h, w2_ref[...], preferred_element_type=jnp.float32) + b2_ref[...]
    pred_ref[...] = 1.0 / (1.0 + jnp.exp(-z))


def _tc_mlp(meta, nla, emb, w1m, w1n, w1e, b1, w2, b2):
    grid = (B // BS,)
    blk = lambda r, c: pl.BlockSpec((r, c), lambda i: (i, 0))
    full = lambda r, c: pl.BlockSpec((r, c), lambda i: (0, 0))
    return pl.pallas_call(
        _mlp_body,
        grid=grid,
        in_specs=[
            blk(BS, 200), blk(BS, 1), blk(BS, NCOL * EDIM),
            full(200, 20), full(1, 20), full(NCOL * EDIM, 20),
            full(1, 20), full(20, 1), full(1, 1),
        ],
        out_specs=[blk(BS, DIN), blk(BS, 1)],
        out_shape=[
            jax.ShapeDtypeStruct((B, DIN), jnp.float32),
            jax.ShapeDtypeStruct((B, 1), jnp.float32),
        ],
    )(meta, nla, emb, w1m, w1n, w1e, b1, w2, b2)


def kernel(meta_features, nla, components, tables, W1, b1, W2, b2):
    tT = tables.transpose(0, 2, 1)
    t128 = _tc_transpose(tT)
    table_flat = t128.reshape(NCOL * VPAD, EDIM)

    comp = components.astype(jnp.int32)
    col_off = (jnp.arange(NCOL, dtype=jnp.int32) * VPAD)[None, :]
    j, vl = comp // VBLK, comp % VBLK
    packed = (j * PR + vl % PR) * 8 + vl // PR
    idx_flat = (packed + col_off).reshape(B_TOT)

    emb_flat = _sc_gather(table_flat, idx_flat)
    emb = emb_flat.reshape(B, NCOL * EDIM)

    w1m = W1[0:200]
    w1n = W1[200:201]
    w1e = W1[201:DIN]
    embedding, pred = _tc_mlp(meta_features, nla, emb, w1m, w1n, w1e,
                              b1.reshape(1, 20), W2, b2.reshape(1, 1))
    return (embedding, pred)


# speedup vs baseline: 4.0082x; 1.0109x over previous
import functools

import jax
import jax.numpy as jnp
from jax import lax
from jax.experimental import pallas as pl
from jax.experimental.pallas import tpu as pltpu
from jax.experimental.pallas import tpu_sc as plsc

B = 16384
NCOL = 26
VOCAB = 100000
EDIM = 16
DIN = 200 + 1 + NCOL * EDIM

NC = 2
NS = 16
NW = NC * NS
B_TOT = B * NCOL
B_PER_W = B_TOT // NW
CHUNK = 1664
NCHUNK = B_PER_W // CHUNK

VBLK = 100352
NBLK = 1
VPAD = NBLK * VBLK
PR = VBLK // 8


def _xpose_body(t_ref, out_ref):
    x = t_ref[0]
    lv = lax.broadcasted_iota(jnp.int32, (EDIM, VBLK), 1) + pl.program_id(1) * VBLK
    x = jnp.where(lv < VOCAB, x, 0.0)
    xs = jnp.concatenate([x[:, s * PR:(s + 1) * PR] for s in range(8)], axis=0)
    out_ref[...] = jnp.transpose(xs)[None]


def _tc_transpose(tT):
    return pl.pallas_call(
        _xpose_body,
        grid=(NCOL, NBLK),
        in_specs=[pl.BlockSpec((1, EDIM, VBLK), lambda c, j: (c, 0, j))],
        out_specs=pl.BlockSpec((1, PR, 128), lambda c, j: (c, j, 0)),
        out_shape=jax.ShapeDtypeStruct((NCOL, NBLK * PR, 128), jnp.float32),
    )(tT)


def _sc_gather(table_flat, idx_flat):
    mesh = plsc.VectorSubcoreMesh(core_axis_name="c", subcore_axis_name="s")

    @functools.partial(
        pl.kernel,
        mesh=mesh,
        out_type=jax.ShapeDtypeStruct((B_TOT, EDIM), jnp.float32),
        scratch_types=[
            pltpu.VMEM((CHUNK,), jnp.int32),
            pltpu.VMEM((CHUNK, EDIM), jnp.float32),
            pltpu.SemaphoreType.DMA,
        ],
        compiler_params=pltpu.CompilerParams(use_tc_tiling_on_sc=False),
    )
    def gather_k(table_hbm, idx_hbm, out_hbm, idx_v, rows_v, sem):
        wid = lax.axis_index("s") * NC + lax.axis_index("c")
        base = wid * B_PER_W
        for i in range(NCHUNK):
            off = base + i * CHUNK
            pltpu.sync_copy(idx_hbm.at[pl.ds(off, CHUNK)], idx_v)
            pltpu.async_copy(table_hbm.at[idx_v], rows_v, sem).wait()
            pltpu.sync_copy(rows_v, out_hbm.at[pl.ds(off, CHUNK)])

    return gather_k(table_flat, idx_flat)


BS = 512


def _mlp_body(meta_ref, nla_ref, emb_ref, w1m_ref, w1n_ref, w1e_ref,
              b1_ref, w2_ref, b2_ref, out_ref, pred_ref):
    m = meta_ref[...]
    n = nla_ref[...]
    e = emb_ref[...]
    out_ref[...] = jnp.concatenate([m, n, e], axis=1)
    h = jnp.dot(m, w1m_ref[...], preferred_element_type=jnp.float32)
    h = h + jnp.dot(e, w1e_ref[...], preferred_element_type=jnp.float32)
    h = h + n * w1n_ref[...]
    h = jnp.maximum(h + b1_ref[...], 0.0)
    z = jnp.dot(h, w2_ref[...], preferred_element_type=jnp.float32) + b2_ref[...]
    pred_ref[...] = 1.0 / (1.0 + jnp.exp(-z))


def _tc_mlp(meta, nla, emb, w1m, w1n, w1e, b1, w2, b2):
    grid = (B // BS,)
    blk = lambda r, c: pl.BlockSpec((r, c), lambda i: (i, 0))
    full = lambda r, c: pl.BlockSpec((r, c), lambda i: (0, 0))
    return pl.pallas_call(
        _mlp_body,
        grid=grid,
        in_specs=[
            blk(BS, 200), blk(BS, 1), blk(BS, NCOL * EDIM),
            full(200, 20), full(1, 20), full(NCOL * EDIM, 20),
            full(1, 20), full(20, 1), full(1, 1),
        ],
        out_specs=[blk(BS, DIN), blk(BS, 1)],
        out_shape=[
            jax.ShapeDtypeStruct((B, DIN), jnp.float32),
            jax.ShapeDtypeStruct((B, 1), jnp.float32),
        ],
    )(meta, nla, emb, w1m, w1n, w1e, b1, w2, b2)


def kernel(meta_features, nla, components, tables, W1, b1, W2, b2):
    tT = tables.transpose(0, 2, 1)
    t128 = _tc_transpose(tT)
    table_flat = t128.reshape(NCOL * VPAD, EDIM)

    comp = components.astype(jnp.int32)
    col_off = (jnp.arange(NCOL, dtype=jnp.int32) * VPAD)[None, :]
    j, vl = comp // VBLK, comp % VBLK
    packed = (j * PR + vl % PR) * 8 + vl // PR
    idx_flat = (packed + col_off).reshape(B_TOT)

    emb_flat = _sc_gather(table_flat, idx_flat)
    emb = emb_flat.reshape(B, NCOL * EDIM)

    w1m = W1[0:200]
    w1n = W1[200:201]
    w1e = W1[201:DIN]
    embedding, pred = _tc_mlp(meta_features, nla, emb, w1m, w1n, w1e,
                              b1.reshape(1, 20), W2, b2.reshape(1, 1))
    return (embedding, pred)


# speedup vs baseline: 4.6154x; 1.1515x over previous
import functools

import jax
import jax.numpy as jnp
from jax import lax
from jax.experimental import pallas as pl
from jax.experimental.pallas import tpu as pltpu
from jax.experimental.pallas import tpu_sc as plsc

B = 16384
NCOL = 26
VOCAB = 100000
EDIM = 16
DIN = 200 + 1 + NCOL * EDIM

NC = 2
NS = 16
NW = NC * NS
B_TOT = B * NCOL
B_PER_W = B_TOT // NW
CHUNK = 1664
NCHUNK = B_PER_W // CHUNK

VBLK = 100352
NBLK = 1
VPAD = NBLK * VBLK
PR = VBLK // 8


def _xpose_body(t_ref, out_ref):
    x = t_ref[0]
    lv = lax.broadcasted_iota(jnp.int32, (EDIM, VBLK), 1) + pl.program_id(1) * VBLK
    x = jnp.where(lv < VOCAB, x, 0.0)
    xs = jnp.concatenate([x[:, s * PR:(s + 1) * PR] for s in range(8)], axis=0)
    out_ref[...] = jnp.transpose(xs)[None]


def _tc_transpose(tT):
    return pl.pallas_call(
        _xpose_body,
        grid=(NCOL, NBLK),
        in_specs=[pl.BlockSpec((1, EDIM, VBLK), lambda c, j: (c, 0, j))],
        out_specs=pl.BlockSpec((1, PR, 128), lambda c, j: (c, j, 0)),
        out_shape=jax.ShapeDtypeStruct((NCOL, NBLK * PR, 128), jnp.float32),
    )(tT)


def _sc_gather(table_flat, idx_flat):
    mesh = plsc.VectorSubcoreMesh(core_axis_name="c", subcore_axis_name="s")

    @functools.partial(
        pl.kernel,
        mesh=mesh,
        out_type=jax.ShapeDtypeStruct((B_TOT, EDIM), jnp.float32),
        scratch_types=[
            pltpu.VMEM((CHUNK,), jnp.int32),
            pltpu.VMEM((CHUNK, EDIM), jnp.float32),
            pltpu.SemaphoreType.DMA,
        ],
        compiler_params=pltpu.CompilerParams(use_tc_tiling_on_sc=False),
    )
    def gather_k(table_hbm, idx_hbm, out_hbm, idx_v, rows_v, sem):
        wid = lax.axis_index("s") * NC + lax.axis_index("c")
        base = wid * B_PER_W
        for i in range(NCHUNK):
            off = base + i * CHUNK
            pltpu.sync_copy(idx_hbm.at[pl.ds(off, CHUNK)], idx_v)
            pltpu.async_copy(table_hbm.at[idx_v], rows_v, sem).wait()
            pltpu.sync_copy(rows_v, out_hbm.at[pl.ds(off, CHUNK)])

    return gather_k(table_flat, idx_flat)


BS = 512


def _mlp_body(metaT_ref, nlaT_ref, embT_ref, w1T_ref, b1_ref, w2T_ref,
              b2_ref, outT_ref, predT_ref):
    m = metaT_ref[...]
    n = nlaT_ref[...]
    e = embT_ref[...]
    cat = jnp.concatenate([m, n, e], axis=0)
    outT_ref[...] = cat
    h = jnp.dot(w1T_ref[...], cat, preferred_element_type=jnp.float32)
    h = jnp.maximum(h + b1_ref[...], 0.0)
    z = jnp.dot(w2T_ref[...], h, preferred_element_type=jnp.float32) + b2_ref[...]
    predT_ref[...] = 1.0 / (1.0 + jnp.exp(-z))


def _tc_mlp(metaT, nlaT, embT, w1T, b1, w2T, b2):
    grid = (B // BS,)
    blk = lambda r: pl.BlockSpec((r, BS), lambda i: (0, i))
    full = lambda r, c: pl.BlockSpec((r, c), lambda i: (0, 0))
    return pl.pallas_call(
        _mlp_body,
        grid=grid,
        in_specs=[
            blk(200), blk(1), blk(NCOL * EDIM),
            full(20, DIN), full(20, 1), full(1, 20), full(1, 1),
        ],
        out_specs=[blk(DIN), blk(1)],
        out_shape=[
            jax.ShapeDtypeStruct((DIN, B), jnp.float32),
            jax.ShapeDtypeStruct((1, B), jnp.float32),
        ],
    )(metaT, nlaT, embT, w1T, b1, w2T, b2)


def kernel(meta_features, nla, components, tables, W1, b1, W2, b2):
    tT = tables.transpose(0, 2, 1)
    t128 = _tc_transpose(tT)
    table_flat = t128.reshape(NCOL * VPAD, EDIM)

    comp = components.astype(jnp.int32)
    col_off = (jnp.arange(NCOL, dtype=jnp.int32) * VPAD)[None, :]
    j, vl = comp // VBLK, comp % VBLK
    packed = (j * PR + vl % PR) * 8 + vl // PR
    idx_flat = (packed + col_off).reshape(B_TOT)

    emb_flat = _sc_gather(table_flat, idx_flat)
    embT = emb_flat.reshape(B, NCOL * EDIM).T

    outT, predT = _tc_mlp(meta_features.T, nla.T, embT, W1.T,
                          b1.reshape(20, 1), W2.T, b2.reshape(1, 1))
    return (outT.T, predT.T)


# speedup vs baseline: 4.7898x; 1.0378x over previous
import functools

import jax
import jax.numpy as jnp
from jax import lax
from jax.experimental import pallas as pl
from jax.experimental.pallas import tpu as pltpu
from jax.experimental.pallas import tpu_sc as plsc

B = 16384
NCOL = 26
VOCAB = 100000
EDIM = 16
DIN = 200 + 1 + NCOL * EDIM

HCOL = NCOL // 2

NC = 2
NS = 16
NW = NC * NS
BT_H = B * HCOL
BPW_H = BT_H // NW
CHUNK = 1664
NCHUNK = BPW_H // CHUNK

VBLK = 100352
PR = VBLK // 8


def _xpose_body(t_ref, out_ref):
    x = t_ref[0]
    lv = lax.broadcasted_iota(jnp.int32, (EDIM, VBLK), 1)
    x = jnp.where(lv < VOCAB, x, 0.0)
    xs = jnp.concatenate([x[:, s * PR:(s + 1) * PR] for s in range(8)], axis=0)
    out_ref[...] = jnp.transpose(xs)[None]


def _make_tc_transpose(c0):
    return pl.pallas_call(
        _xpose_body,
        grid=(HCOL,),
        in_specs=[pl.BlockSpec((1, EDIM, VBLK), lambda c: (c0 + c, 0, 0))],
        out_specs=pl.BlockSpec((1, PR, 128), lambda c: (c, 0, 0)),
        out_shape=jax.ShapeDtypeStruct((HCOL, PR, 128), jnp.float32),
    )


def _sc_gather(table_flat, idx_flat):
    mesh = plsc.VectorSubcoreMesh(core_axis_name="c", subcore_axis_name="s")

    @functools.partial(
        pl.kernel,
        mesh=mesh,
        out_type=jax.ShapeDtypeStruct((BT_H, EDIM), jnp.float32),
        scratch_types=[
            pltpu.VMEM((CHUNK,), jnp.int32),
            pltpu.VMEM((CHUNK,), jnp.int32),
            pltpu.VMEM((CHUNK, EDIM), jnp.float32),
            pltpu.VMEM((CHUNK, EDIM), jnp.float32),
            pltpu.SemaphoreType.DMA,
            pltpu.SemaphoreType.DMA,
            pltpu.SemaphoreType.DMA,
        ],
        compiler_params=pltpu.CompilerParams(use_tc_tiling_on_sc=False),
    )
    def gather_k(table_hbm, idx_hbm, out_hbm, ia, ib, ra, rb, gs, wsa, wsb):
        wid = lax.axis_index("s") * NC + lax.axis_index("c")
        base = wid * BPW_H
        idxv = [ia, ib]
        rows = [ra, rb]
        wsem = [wsa, wsb]
        wb = [None, None]
        pltpu.sync_copy(idx_hbm.at[pl.ds(base, CHUNK)], ia)
        for i in range(NCHUNK):
            p = i & 1
            if wb[p] is not None:
                wb[p].wait()
            g = pltpu.async_copy(table_hbm.at[idxv[p]], rows[p], gs)
            if i + 1 < NCHUNK:
                pltpu.sync_copy(
                    idx_hbm.at[pl.ds(base + (i + 1) * CHUNK, CHUNK)],
                    idxv[1 - p])
            g.wait()
            wb[p] = pltpu.async_copy(
                rows[p], out_hbm.at[pl.ds(base + i * CHUNK, CHUNK)], wsem[p])
        wb[0].wait()
        wb[1].wait()

    return gather_k(table_flat, idx_flat)


BS = 512


def _mlp_body(metaT_ref, nlaT_ref, e1_ref, e2_ref, w1T_ref, b1_ref, w2T_ref,
              b2_ref, outT_ref, predT_ref):
    m = metaT_ref[...]
    n = nlaT_ref[...]
    e1 = e1_ref[...]
    e2 = e2_ref[...]
    cat = jnp.concatenate([m, n, e1, e2], axis=0)
    outT_ref[...] = cat
    h = jnp.dot(w1T_ref[...], cat, preferred_element_type=jnp.float32)
    h = jnp.maximum(h + b1_ref[...], 0.0)
    z = jnp.dot(w2T_ref[...], h, preferred_element_type=jnp.float32) + b2_ref[...]
    predT_ref[...] = 1.0 / (1.0 + jnp.exp(-z))


def _tc_mlp(metaT, nlaT, embT1, embT2, w1T, b1, w2T, b2):
    grid = (B // BS,)
    blk = lambda r: pl.BlockSpec((r, BS), lambda i: (0, i))
    full = lambda r, c: pl.BlockSpec((r, c), lambda i: (0, 0))
    return pl.pallas_call(
        _mlp_body,
        grid=grid,
        in_specs=[
            blk(200), blk(1), blk(HCOL * EDIM), blk(HCOL * EDIM),
            full(20, DIN), full(20, 1), full(1, 20), full(1, 1),
        ],
        out_specs=[blk(DIN), blk(1)],
        out_shape=[
            jax.ShapeDtypeStruct((DIN, B), jnp.float32),
            jax.ShapeDtypeStruct((1, B), jnp.float32),
        ],
    )(metaT, nlaT, embT1, embT2, w1T, b1, w2T, b2)


def _half_idx(comp_half):
    col_off = (jnp.arange(HCOL, dtype=jnp.int32) * VBLK)[None, :]
    packed = (comp_half % PR) * 8 + comp_half // PR
    return (packed + col_off).reshape(BT_H)


def kernel(meta_features, nla, components, tables, W1, b1, W2, b2):
    tT = tables.transpose(0, 2, 1)
    comp = components.astype(jnp.int32)

    t1 = _make_tc_transpose(0)(tT).reshape(HCOL * VBLK, EDIM)
    idx1 = _half_idx(comp[:, :HCOL])
    emb1 = _sc_gather(t1, idx1)

    t2 = _make_tc_transpose(HCOL)(tT).reshape(HCOL * VBLK, EDIM)
    idx2 = _half_idx(comp[:, HCOL:])
    emb2 = _sc_gather(t2, idx2)

    embT1 = emb1.reshape(B, HCOL * EDIM).T
    embT2 = emb2.reshape(B, HCOL * EDIM).T

    outT, predT = _tc_mlp(meta_features.T, nla.T, embT1, embT2, W1.T,
                          b1.reshape(20, 1), W2.T, b2.reshape(1, 1))
    return (outT.T, predT.T)


# speedup vs baseline: 5.3767x; 1.1225x over previous
import functools

import jax
import jax.numpy as jnp
from jax import lax
from jax.experimental import pallas as pl
from jax.experimental.pallas import tpu as pltpu
from jax.experimental.pallas import tpu_sc as plsc

B = 16384
NCOL = 26
VOCAB = 100000
EDIM = 16
DIN = 200 + 1 + NCOL * EDIM

HCOL = NCOL // 2

NC = 2
NS = 16
NW = NC * NS
BT_H = B * HCOL
BPW_H = BT_H // NW
CHUNK = 1664
NCHUNK = BPW_H // CHUNK

VBLK = 100352
PR = VBLK // 8


def _xpose_body(t_ref, out_ref):
    x = t_ref[0]
    lv = lax.broadcasted_iota(jnp.int32, (EDIM, VBLK), 1)
    x = jnp.where(lv < VOCAB, x, 0.0)
    xs = jnp.concatenate([x[:, s * PR:(s + 1) * PR] for s in range(8)], axis=0)
    out_ref[...] = jnp.transpose(xs)[None]


def _make_tc_transpose(c0):
    return pl.pallas_call(
        _xpose_body,
        grid=(HCOL,),
        in_specs=[pl.BlockSpec((1, EDIM, VBLK), lambda c: (c0 + c, 0, 0))],
        out_specs=pl.BlockSpec((1, PR, 128), lambda c: (c, 0, 0)),
        out_shape=jax.ShapeDtypeStruct((HCOL, PR, 128), jnp.float32),
    )


BW = B // NW


def _sc_gather(table_flat, comp_T, c0):
    mesh = plsc.VectorSubcoreMesh(core_axis_name="c", subcore_axis_name="s")

    @functools.partial(
        pl.kernel,
        mesh=mesh,
        out_type=jax.ShapeDtypeStruct((BT_H, EDIM), jnp.float32),
        scratch_types=[
            pltpu.VMEM((HCOL, BW), jnp.int32),
            pltpu.VMEM((BPW_H,), jnp.int32),
            pltpu.VMEM((CHUNK, EDIM), jnp.float32),
            pltpu.VMEM((CHUNK, EDIM), jnp.float32),
            pltpu.SemaphoreType.DMA,
            pltpu.SemaphoreType.DMA,
            pltpu.SemaphoreType.DMA,
        ],
        compiler_params=pltpu.CompilerParams(use_tc_tiling_on_sc=False,
                                             needs_layout_passes=False),
    )
    def gather_k(table_hbm, ct_hbm, out_hbm, ctv, ilist, ra, rb, gs, wsa, wsb):
        wid = lax.axis_index("s") * NC + lax.axis_index("c")
        base = wid * BPW_H
        pltpu.sync_copy(
            ct_hbm.at[pl.ds(c0, HCOL), pl.ds(wid * BW, BW)], ctv)

        lane13 = lax.iota(jnp.int32, 16) * HCOL

        for c in range(HCOL):
            @pl.loop(0, BW // 16)
            def build(g, c=c):
                v = ctv[c, pl.ds(g * 16, 16)]
                plsc.store_scatter(ilist, [g * (16 * HCOL) + c + lane13], v)

        rows = [ra, rb]
        wsem = [wsa, wsb]
        wb = [None, None]
        for i in range(NCHUNK):
            p = i & 1
            if wb[p] is not None:
                wb[p].wait()
            g = pltpu.async_copy(
                table_hbm.at[ilist.at[pl.ds(i * CHUNK, CHUNK)]], rows[p], gs)
            g.wait()
            wb[p] = pltpu.async_copy(
                rows[p], out_hbm.at[pl.ds(base + i * CHUNK, CHUNK)], wsem[p])
        wb[0].wait()
        wb[1].wait()

    return gather_k(table_flat, comp_T)


BS = 1024


def _mlp_body(metaT_ref, nlaT_ref, e1_ref, e2_ref, w1T_ref, b1_ref, w2T_ref,
              b2_ref, outT_ref, predT_ref):
    m = metaT_ref[...]
    n = nlaT_ref[...]
    e1 = e1_ref[...]
    e2 = e2_ref[...]
    cat = jnp.concatenate([m, n, e1, e2], axis=0)
    outT_ref[...] = cat
    h = jnp.dot(w1T_ref[...], cat, preferred_element_type=jnp.float32)
    h = jnp.maximum(h + b1_ref[...], 0.0)
    z = jnp.dot(w2T_ref[...], h, preferred_element_type=jnp.float32) + b2_ref[...]
    predT_ref[...] = 1.0 / (1.0 + jnp.exp(-z))


def _tc_mlp(metaT, nlaT, embT1, embT2, w1T, b1, w2T, b2):
    grid = (B // BS,)
    blk = lambda r: pl.BlockSpec((r, BS), lambda i: (0, i))
    full = lambda r, c: pl.BlockSpec((r, c), lambda i: (0, 0))
    return pl.pallas_call(
        _mlp_body,
        grid=grid,
        in_specs=[
            blk(200), blk(1), blk(HCOL * EDIM), blk(HCOL * EDIM),
            full(20, DIN), full(20, 1), full(1, 20), full(1, 1),
        ],
        out_specs=[blk(DIN), blk(1)],
        out_shape=[
            jax.ShapeDtypeStruct((DIN, B), jnp.float32),
            jax.ShapeDtypeStruct((1, B), jnp.float32),
        ],
    )(metaT, nlaT, embT1, embT2, w1T, b1, w2T, b2)


def kernel(meta_features, nla, components, tables, W1, b1, W2, b2):
    tT = tables.transpose(0, 2, 1)
    comp_T = components.astype(jnp.int32).T
    colv = ((jnp.arange(NCOL, dtype=jnp.int32) % HCOL) * VBLK)[:, None]
    packed_T = (comp_T % PR) * 8 + comp_T // PR + colv

    t1 = _make_tc_transpose(0)(tT).reshape(HCOL * VBLK, EDIM)
    emb1 = _sc_gather(t1, packed_T, 0)

    t2 = _make_tc_transpose(HCOL)(tT).reshape(HCOL * VBLK, EDIM)
    emb2 = _sc_gather(t2, packed_T, HCOL)

    embT1 = emb1.reshape(B, HCOL * EDIM).T
    embT2 = emb2.reshape(B, HCOL * EDIM).T

    outT, predT = _tc_mlp(meta_features.T, nla.T, embT1, embT2, W1.T,
                          b1.reshape(20, 1), W2.T, b2.reshape(1, 1))
    return (outT.T, predT.T)
